# baseline (device time: 67873 ns/iter reference)
import jax
import jax.numpy as jnp
from jax import lax
from jax.experimental import pallas as pl
from jax.experimental.pallas import tpu as pltpu

HALF_M = 512
HALF_F = 2048


def kernel(x, dy):
    k, m = x.shape
    _, f = dy.shape

    def body(x_ref, dy_ref, out_ref,
             ysend, yrecv, xsend, xrecv,
             ysend_sem, yrecv_sem, xsend_sem, xrecv_sem):
        my_x = lax.axis_index("x")
        my_y = lax.axis_index("y")
        other_x = 1 - my_x
        other_y = 1 - my_y

        barrier_sem = pltpu.get_barrier_semaphore()
        pl.semaphore_signal(barrier_sem, inc=1, device_id=(other_x, my_y),
                            device_id_type=pl.DeviceIdType.MESH)
        pl.semaphore_signal(barrier_sem, inc=1, device_id=(my_x, other_y),
                            device_id_type=pl.DeviceIdType.MESH)
        pl.semaphore_wait(barrier_sem, 2)

        dyb = dy_ref[:, pl.ds(my_x * HALF_F, HALF_F)].astype(jnp.bfloat16)

        xs = x_ref[:, pl.ds(other_y * HALF_M, HALF_M)].astype(jnp.bfloat16)
        p_send = lax.dot_general(
            xs, dyb, (((0,), (0,)), ((), ())),
            preferred_element_type=jnp.float32)
        ysend[...] = p_send.astype(jnp.bfloat16)

        y_rdma = pltpu.make_async_remote_copy(
            src_ref=ysend, dst_ref=yrecv,
            send_sem=ysend_sem, recv_sem=yrecv_sem,
            device_id=(my_x, other_y), device_id_type=pl.DeviceIdType.MESH)
        y_rdma.start()

        xk = x_ref[:, pl.ds(my_y * HALF_M, HALF_M)].astype(jnp.bfloat16)
        p_keep = lax.dot_general(
            xk, dyb, (((0,), (0,)), ((), ())),
            preferred_element_type=jnp.float32)

        y_rdma.wait()
        r = p_keep + yrecv[...].astype(jnp.float32)
        xsend[...] = r.astype(jnp.bfloat16)

        x_rdma = pltpu.make_async_remote_copy(
            src_ref=xsend, dst_ref=xrecv,
            send_sem=xsend_sem, recv_sem=xrecv_sem,
            device_id=(other_x, my_y), device_id_type=pl.DeviceIdType.MESH)
        x_rdma.start()

        out_ref[:, pl.ds(my_x * HALF_F, HALF_F)] = r
        x_rdma.wait()
        out_ref[:, pl.ds(other_x * HALF_F, HALF_F)] = xrecv[...].astype(jnp.float32)

    return pl.pallas_call(
        body,
        out_shape=jax.ShapeDtypeStruct((HALF_M, f), jnp.float32),
        in_specs=[pl.BlockSpec(memory_space=pltpu.VMEM),
                  pl.BlockSpec(memory_space=pltpu.VMEM)],
        out_specs=pl.BlockSpec(memory_space=pltpu.VMEM),
        scratch_shapes=[
            pltpu.VMEM((HALF_M, HALF_F), jnp.bfloat16),
            pltpu.VMEM((HALF_M, HALF_F), jnp.bfloat16),
            pltpu.VMEM((HALF_M, HALF_F), jnp.bfloat16),
            pltpu.VMEM((HALF_M, HALF_F), jnp.bfloat16),
            pltpu.SemaphoreType.DMA,
            pltpu.SemaphoreType.DMA,
            pltpu.SemaphoreType.DMA,
            pltpu.SemaphoreType.DMA,
        ],
        compiler_params=pltpu.CompilerParams(collective_id=0),
    )(x, dy)


# device time: 45573 ns/iter; 1.4893x vs baseline; 1.4893x over previous
import jax
import jax.numpy as jnp
from jax import lax
from jax.experimental import pallas as pl
from jax.experimental.pallas import tpu as pltpu

HALF_M = 512
HALF_F = 2048
NCHUNK = 8
CH = HALF_F // NCHUNK


def kernel(x, dy):
    k, m = x.shape
    _, f = dy.shape

    def body(x_ref, dy_ref, out_ref,
             ysend, yrecv, xsend, xrecv,
             ysend_sems, yrecv_sems, xsend_sems, xrecv_sems):
        my_x = lax.axis_index("x")
        my_y = lax.axis_index("y")
        other_x = 1 - my_x
        other_y = 1 - my_y
        col0 = my_x * HALF_F

        barrier_sem = pltpu.get_barrier_semaphore()
        pl.semaphore_signal(barrier_sem, inc=1, device_id=(other_x, my_y),
                            device_id_type=pl.DeviceIdType.MESH)
        pl.semaphore_signal(barrier_sem, inc=1, device_id=(my_x, other_y),
                            device_id_type=pl.DeviceIdType.MESH)
        pl.semaphore_wait(barrier_sem, 2)

        xs_b = x_ref[:, pl.ds(other_y * HALF_M, HALF_M)].astype(jnp.bfloat16)
        xk_b = x_ref[:, pl.ds(my_y * HALF_M, HALF_M)].astype(jnp.bfloat16)

        def dy_chunk(c):
            return dy_ref[:, pl.ds(col0 + c * CH, CH)].astype(jnp.bfloat16)

        y_rdmas = []
        for c in range(NCHUNK):
            ps = lax.dot_general(
                xs_b, dy_chunk(c), (((0,), (0,)), ((), ())),
                preferred_element_type=jnp.float32)
            ysend[c] = ps.astype(jnp.bfloat16)
            rdma = pltpu.make_async_remote_copy(
                src_ref=ysend.at[c], dst_ref=yrecv.at[c],
                send_sem=ysend_sems.at[c], recv_sem=yrecv_sems.at[c],
                device_id=(my_x, other_y),
                device_id_type=pl.DeviceIdType.MESH)
            rdma.start()
            y_rdmas.append(rdma)

        x_rdmas = []
        for c in range(NCHUNK):
            pk = lax.dot_general(
                xk_b, dy_chunk(c), (((0,), (0,)), ((), ())),
                preferred_element_type=jnp.float32)
            y_rdmas[c].wait()
            r = pk + yrecv[c].astype(jnp.float32)
            xsend[c] = r.astype(jnp.bfloat16)
            rdma = pltpu.make_async_remote_copy(
                src_ref=xsend.at[c], dst_ref=xrecv.at[c],
                send_sem=xsend_sems.at[c], recv_sem=xrecv_sems.at[c],
                device_id=(other_x, my_y),
                device_id_type=pl.DeviceIdType.MESH)
            rdma.start()
            x_rdmas.append(rdma)
            out_ref[:, pl.ds(col0 + c * CH, CH)] = r

        for c in range(NCHUNK):
            x_rdmas[c].wait()
            out_ref[:, pl.ds(other_x * HALF_F + c * CH, CH)] = (
                xrecv[c].astype(jnp.float32))

    return pl.pallas_call(
        body,
        out_shape=jax.ShapeDtypeStruct((HALF_M, f), jnp.float32),
        in_specs=[pl.BlockSpec(memory_space=pltpu.VMEM),
                  pl.BlockSpec(memory_space=pltpu.VMEM)],
        out_specs=pl.BlockSpec(memory_space=pltpu.VMEM),
        scratch_shapes=[
            pltpu.VMEM((NCHUNK, HALF_M, CH), jnp.bfloat16),
            pltpu.VMEM((NCHUNK, HALF_M, CH), jnp.bfloat16),
            pltpu.VMEM((NCHUNK, HALF_M, CH), jnp.bfloat16),
            pltpu.VMEM((NCHUNK, HALF_M, CH), jnp.bfloat16),
            pltpu.SemaphoreType.DMA((NCHUNK,)),
            pltpu.SemaphoreType.DMA((NCHUNK,)),
            pltpu.SemaphoreType.DMA((NCHUNK,)),
            pltpu.SemaphoreType.DMA((NCHUNK,)),
        ],
        compiler_params=pltpu.CompilerParams(collective_id=0),
    )(x, dy)
